# Initial kernel scaffold; baseline (speedup 1.0000x reference)
import jax
import jax.numpy as jnp
from jax.experimental import pallas as pl


def kernel(x, edge_index_quote, edge_index_reply, edge_index_mention, edge_weight_quote, edge_weight_reply, edge_weight_mention, params):
    raise NotImplementedError("write your pallas kernel here")


# speedup vs baseline: 10.6363x; 10.6363x over previous
import functools

import jax
import jax.numpy as jnp
from jax import lax
from jax.experimental import pallas as pl
from jax.experimental.pallas import tpu as pltpu
from jax.experimental.pallas import tpu_sc as plsc

_N = 10000
_DIN = 128
_DH = 64
_E = 320000
_N3 = 3 * _N
_E3 = 3 * _E
_NW = 32
_EPW = _E // _NW
_C = 400
_NCHUNK = _EPW // _C
_SUB = 80
_NSUB = _C // _SUB
_NTILE = 16
_STRIPE = _N // _NTILE


def _proj_tail(r, hp, asrc_ref, adst_ref, hp_ref, ssrc_ref, sdst_ref, m_ref):
    hp_ref[0] = hp
    ssrc = jnp.sum(hp * asrc_ref[0, 0][None, :], axis=1)
    sdst = jnp.sum(hp * adst_ref[0, 0][None, :], axis=1)
    ssrc_ref[0, 0] = ssrc
    sdst_ref[0, 0] = sdst
    z = jnp.max(ssrc) + jnp.max(sdst)
    lm = jnp.maximum(z, 0.2 * z)

    @pl.when(r == 0)
    def _():
        m_ref[0, 0] = lm

    @pl.when(r != 0)
    def _():
        m_ref[0, 0] = jnp.maximum(m_ref[0, 0], lm)


def _proj0_body(x_ref, w_ref, asrc_ref, adst_ref,
                hp_ref, ssrc_ref, sdst_ref, m_ref):
    r = pl.program_id(0)
    hp = jnp.dot(x_ref[...], w_ref[0], preferred_element_type=jnp.float32)
    _proj_tail(r, hp, asrc_ref, adst_ref, hp_ref, ssrc_ref, sdst_ref, m_ref)


def _finish(raw_ref, den_ref, b_ref):
    raw = raw_ref[0, 0] + raw_ref[1, 0]
    den = jnp.sum(den_ref[:, 0, 0, :], axis=0)
    return raw / (den + 1e-16)[:, None] + b_ref[0, 0][None, :]


def _finproj_body(raw_ref, den_ref, b_ref, w_ref, asrc_ref, adst_ref,
                  hp_ref, ssrc_ref, sdst_ref, m_ref):
    r = pl.program_id(0)
    h = jnp.maximum(_finish(raw_ref, den_ref, b_ref), 0.0)
    hp = jnp.dot(h, w_ref[0], preferred_element_type=jnp.float32)
    _proj_tail(r, hp, asrc_ref, adst_ref, hp_ref, ssrc_ref, sdst_ref, m_ref)


def _finlast_body(raw_ref, den_ref, b_ref, out_ref):
    out_ref[0] = _finish(raw_ref, den_ref, b_ref)


_PROJ_OUT_SHAPE = [
    jax.ShapeDtypeStruct((3, _N, _DH), jnp.float32),
    jax.ShapeDtypeStruct((3, 1, _N), jnp.float32),
    jax.ShapeDtypeStruct((3, 1, _N), jnp.float32),
    jax.ShapeDtypeStruct((1, 1), jnp.float32),
]
_PROJ_OUT_SPECS = [
    pl.BlockSpec((1, _N, _DH), lambda r: (r, 0, 0)),
    pl.BlockSpec((1, 1, _N), lambda r: (r, 0, 0)),
    pl.BlockSpec((1, 1, _N), lambda r: (r, 0, 0)),
    pl.BlockSpec(memory_space=pltpu.SMEM),
]
_W_SPEC0 = pl.BlockSpec((1, _DIN, _DH), lambda r: (r, 0, 0))
_W_SPEC = pl.BlockSpec((1, _DH, _DH), lambda r: (r, 0, 0))
_A_SPEC = pl.BlockSpec((1, 1, _DH), lambda r: (r, 0, 0))
_RAW_SPEC = pl.BlockSpec((2, 1, _N, _DH), lambda r: (0, r, 0, 0))
_DEN_SPEC = pl.BlockSpec((_NW, 1, 1, _N), lambda r: (0, r, 0, 0))


def _proj0(x, w, asrc, adst):
    return pl.pallas_call(
        _proj0_body,
        grid=(3,),
        in_specs=[
            pl.BlockSpec((_N, _DIN), lambda r: (0, 0)),
            _W_SPEC0, _A_SPEC, _A_SPEC,
        ],
        out_specs=_PROJ_OUT_SPECS,
        out_shape=_PROJ_OUT_SHAPE,
    )(x, w, asrc, adst)


def _finproj(raw, den, b, w, asrc, adst):
    return pl.pallas_call(
        _finproj_body,
        grid=(3,),
        in_specs=[_RAW_SPEC, _DEN_SPEC, _A_SPEC, _W_SPEC, _A_SPEC, _A_SPEC],
        out_specs=_PROJ_OUT_SPECS,
        out_shape=_PROJ_OUT_SHAPE,
    )(raw, den, b, w, asrc, adst)


def _finlast(raw, den, b):
    return pl.pallas_call(
        _finlast_body,
        grid=(3,),
        in_specs=[_RAW_SPEC, _DEN_SPEC, _A_SPEC],
        out_specs=pl.BlockSpec((1, _N, _DH), lambda r: (r, 0, 0)),
        out_shape=jax.ShapeDtypeStruct((3, _N, _DH), jnp.float32),
    )(raw, den, b)


def _edge_body(src1_h, dst1_h, ew1_h, hp_h, ssrc_h, sdst_h,
               m_h, raw_o, den_o,
               ssrc_v, sdst_v, den_v, srcf, dstf, ewf, dst2, rows, m_v,
               out_sh, sem):
    cid = lax.axis_index("c")
    sid = lax.axis_index("s")
    wid = cid * _NTILE + sid
    zeros16 = jnp.zeros((16,), jnp.float32)

    pltpu.sync_copy(ssrc_h, ssrc_v)
    pltpu.sync_copy(sdst_h, sdst_v)
    pltpu.sync_copy(m_h, m_v)

    def _zden(i, carry):
        den_v[pl.ds(i * 16, 16)] = zeros16
        return carry

    lax.fori_loop(0, _N // 16, _zden, 0)

    def _zrow(i, carry):
        for c4 in range(_DH // 16):
            rows[i, pl.ds(c4 * 16, 16)] = zeros16
        return carry

    lax.fori_loop(0, _C, _zrow, 0)
    pltpu.sync_copy(rows.at[pl.ds(0, _C)],
                    out_sh.at[pl.ds(sid * _STRIPE, _C)])
    pltpu.sync_copy(rows.at[pl.ds(0, _STRIPE - _C)],
                    out_sh.at[pl.ds(sid * _STRIPE + _C, _STRIPE - _C)])
    plsc.subcore_barrier()

    mvec = m_v[...]
    iota16 = lax.iota(jnp.int32, 16)

    def _chunk(k, carry):
        base = wid * _EPW + k * _C
        pltpu.sync_copy(src1_h.at[pl.ds(base, _C)], srcf)
        pltpu.sync_copy(dst1_h.at[pl.ds(base, _C)], dstf)
        pltpu.sync_copy(ew1_h.at[pl.ds(base, _C)], ewf)
        for j in range(_NSUB):
            for g in range(_SUB // 16):
                dst2[j, pl.ds(g * 16, 16)] = dstf[pl.ds(j * _SUB + g * 16, 16)]
        cps = [pltpu.async_copy(hp_h.at[srcf.at[pl.ds(j * _SUB, _SUB)]],
                                rows.at[pl.ds(j * _SUB, _SUB)], sem)
               for j in range(_NSUB)]
        for cp in cps:
            cp.wait()

        def _grp(jg, c2):
            sl = pl.ds(jg * 16, 16)
            si = srcf[sl]
            di = dstf[sl]
            z = plsc.load_gather(ssrc_v, [si]) + plsc.load_gather(sdst_v, [di])
            l = jnp.maximum(z, 0.2 * z)
            p = jnp.exp(l - mvec)
            plsc.addupdate_scatter(den_v, [di], p)
            q = p * ewf[sl]
            ei = iota16 + jg * 16
            for c in range(_DH):
                ci = jnp.full((16,), c, jnp.int32)
                col = plsc.load_gather(rows, [ei, ci])
                plsc.store_scatter(rows, [ei, ci], col * q)
            return c2

        lax.fori_loop(0, _C // 16, _grp, 0)
        for j in range(_NSUB):
            pltpu.sync_copy(rows.at[pl.ds(j * _SUB, _SUB)],
                            out_sh.at[dst2.at[j]], add=True)
        return carry

    lax.fori_loop(0, _NCHUNK, _chunk, 0)
    plsc.subcore_barrier()

    pltpu.sync_copy(den_v, den_o.at[wid, 0])
    pltpu.sync_copy(out_sh.at[pl.ds(sid * _STRIPE, _STRIPE)],
                    raw_o.at[cid, pl.ds(sid * _STRIPE, _STRIPE)])


def _edge(src1, dst1, ew1, hp, ssrc, sdst, mvec):
    mesh = plsc.VectorSubcoreMesh(core_axis_name="c", subcore_axis_name="s")
    kern = pl.kernel(
        _edge_body,
        mesh=mesh,
        compiler_params=pltpu.CompilerParams(needs_layout_passes=False, use_tc_tiling_on_sc=False),
        out_type=[
            jax.ShapeDtypeStruct((2, _N, _DH), jnp.float32),
            jax.ShapeDtypeStruct((_NW, 1, _N), jnp.float32),
        ],
        scratch_types=[
            pltpu.VMEM((_N,), jnp.float32),
            pltpu.VMEM((_N,), jnp.float32),
            pltpu.VMEM((_N,), jnp.float32),
            pltpu.VMEM((_C,), jnp.int32),
            pltpu.VMEM((_C,), jnp.int32),
            pltpu.VMEM((_C,), jnp.float32),
            pltpu.VMEM((_NSUB, _SUB), jnp.int32),
            pltpu.VMEM((_C, _DH), jnp.float32),
            pltpu.VMEM((16,), jnp.float32),
            pltpu.VMEM_SHARED((_N, _DH), jnp.float32),
            pltpu.SemaphoreType.DMA,
        ],
    )
    return kern(src1, dst1, ew1, hp, ssrc, sdst, mvec)


def kernel(x, edge_index_quote, edge_index_reply, edge_index_mention,
           edge_weight_quote, edge_weight_reply, edge_weight_mention, params):
    rels = ("quote", "reply", "mention")
    edges = (edge_index_quote, edge_index_reply, edge_index_mention)
    ews = (edge_weight_quote, edge_weight_reply, edge_weight_mention)

    def edge_layer(hp, ssrc, sdst, m):
        mv = jnp.broadcast_to(m.reshape(()), (16,))
        raws, dens = [], []
        for r in range(3):
            raw_r, den_r = _edge(edges[r][0], edges[r][1], ews[r],
                                 hp[r], ssrc[r, 0], sdst[r, 0], mv)
            raws.append(raw_r)
            dens.append(den_r)
        return (jnp.stack(raws, axis=1).reshape(2, 3, _N, _DH),
                jnp.stack(dens, axis=1).reshape(_NW, 3, 1, _N))
    Ws = [jnp.stack([params[r][li]["W"] for r in rels]) for li in range(3)]
    Asrc = [jnp.stack([params[r][li]["a_src"] for r in rels]).reshape(3, 1, _DH)
            for li in range(3)]
    Adst = [jnp.stack([params[r][li]["a_dst"] for r in rels]).reshape(3, 1, _DH)
            for li in range(3)]
    Bs = [jnp.stack([params[r][li]["b"] for r in rels]).reshape(3, 1, _DH)
          for li in range(3)]

    hp, ssrc, sdst, m = _proj0(x, Ws[0], Asrc[0], Adst[0])
    for li in (1, 2):
        raw, den = edge_layer(hp, ssrc, sdst, m)
        hp, ssrc, sdst, m = _finproj(raw, den, Bs[li - 1],
                                     Ws[li], Asrc[li], Adst[li])
    raw, den = edge_layer(hp, ssrc, sdst, m)
    out3 = _finlast(raw, den, Bs[2])
    return jnp.concatenate([out3[0], out3[1], out3[2]], axis=1)
